# baseline (speedup 1.0000x reference)
import functools

import jax
import jax.numpy as jnp
from jax.experimental import pallas as pl
from jax.experimental.pallas import tpu as pltpu


def _pool(y, hp_ref, *, B, H, W, Cout):
    y = y.reshape(B * (H // 2), 2, W, Cout)
    hp = jnp.maximum(y[:, 0], y[:, 1])
    R = B * (H // 2) * W
    hp_ref[...] = hp.reshape(R, Cout)
    pooled = jnp.maximum(hp_ref[pl.ds(0, R // 2, 2), :],
                         hp_ref[pl.ds(1, R // 2, 2), :])
    return pooled.reshape(B, H // 2, W // 2, Cout)


def _conv_taps(x_view, w_ref, shift_ref, *, C, taps):
    y = shift_ref[...].astype(jnp.float32)
    for k in range(taps):
        tap = x_view(k)
        y = y + jnp.dot(tap, w_ref[k * C:(k + 1) * C, :],
                        preferred_element_type=jnp.float32)
    return jnp.maximum(y, 0.0)


def _convs_kernel(p_ref, w1_ref, s1_ref, w2_ref, s2_ref, w3_ref, s3_ref,
                  out_ref, pad2_ref, pad3_ref, hp1_ref, hp2_ref, hp3_ref,
                  *, B, H1, KC, C1, C2, C3):
    H2, H3 = H1 // 2, H1 // 4
    @pl.when(pl.program_id(0) == 0)
    def _():
        pad2_ref[...] = jnp.zeros_like(pad2_ref)
        pad3_ref[...] = jnp.zeros_like(pad3_ref)

    y1 = _conv_taps(
        lambda dy: p_ref[:, dy:dy + H1, :, :].reshape(B * H1 * H1, KC),
        w1_ref, s1_ref, C=KC, taps=3)
    p1 = _pool(y1, hp1_ref, B=B, H=H1, W=H1, Cout=C1)
    pad2_ref[:, 1:H2 + 1, 1:H2 + 1, :] = p1.astype(jnp.bfloat16)

    y2 = _conv_taps(
        lambda k: pad2_ref[:, k // 3:k // 3 + H2, k % 3:k % 3 + H2, :].reshape(
            B * H2 * H2, C1),
        w2_ref, s2_ref, C=C1, taps=9)
    p2 = _pool(y2, hp2_ref, B=B, H=H2, W=H2, Cout=C2)
    pad3_ref[:, 1:H3 + 1, 1:H3 + 1, :] = p2.astype(jnp.bfloat16)

    y3 = _conv_taps(
        lambda k: pad3_ref[:, k // 3:k // 3 + H3, k % 3:k % 3 + H3, :].reshape(
            B * H3 * H3, C2),
        w3_ref, s3_ref, C=C2, taps=9)
    p3 = _pool(y3, hp3_ref, B=B, H=H3, W=H3, Cout=C3)
    out_ref[...] = p3.astype(jnp.bfloat16)


def _convs(rowp, w1, s1, w2, s2, w3, s3, *, B):
    N, Hp, W, KC = rowp.shape
    H1 = Hp - 2
    C1, C2, C3 = w1.shape[1], w2.shape[1], w3.shape[1]
    H2, H3, H4 = H1 // 2, H1 // 4, H1 // 8
    body = functools.partial(_convs_kernel, B=B, H1=H1, KC=KC,
                             C1=C1, C2=C2, C3=C3)
    return pl.pallas_call(
        body,
        out_shape=jax.ShapeDtypeStruct((N, H4, H4, C3), jnp.bfloat16),
        grid_spec=pltpu.PrefetchScalarGridSpec(
            num_scalar_prefetch=0,
            grid=(N // B,),
            in_specs=[
                pl.BlockSpec((B, Hp, W, KC), lambda n: (n, 0, 0, 0)),
                pl.BlockSpec((3 * KC, C1), lambda n: (0, 0)),
                pl.BlockSpec((1, C1), lambda n: (0, 0)),
                pl.BlockSpec((9 * C1, C2), lambda n: (0, 0)),
                pl.BlockSpec((1, C2), lambda n: (0, 0)),
                pl.BlockSpec((9 * C2, C3), lambda n: (0, 0)),
                pl.BlockSpec((1, C3), lambda n: (0, 0)),
            ],
            out_specs=pl.BlockSpec((B, H4, H4, C3), lambda n: (n, 0, 0, 0)),
            scratch_shapes=[
                pltpu.VMEM((B, H2 + 2, H2 + 2, C1), jnp.bfloat16),
                pltpu.VMEM((B, H3 + 2, H3 + 2, C2), jnp.bfloat16),
                pltpu.VMEM((B * (H1 // 2) * H1, C1), jnp.float32),
                pltpu.VMEM((B * (H2 // 2) * H2, C2), jnp.float32),
                pltpu.VMEM((B * (H3 // 2) * H3, C3), jnp.float32),
            ],
        ),
        compiler_params=pltpu.CompilerParams(
            dimension_semantics=("parallel",),
            vmem_limit_bytes=110 * 1024 * 1024),
    )(rowp, w1, s1, w2, s2, w3, s3)


def _mlp_kernel(x_ref, w1_ref, b1_ref, w2_ref, b2_ref, o_ref):
    h = jnp.dot(x_ref[...], w1_ref[...], preferred_element_type=jnp.float32)
    h = jnp.maximum(h + b1_ref[...], 0.0).astype(jnp.bfloat16)
    o_ref[...] = (jnp.dot(h, w2_ref[...], preferred_element_type=jnp.float32)
                  + b2_ref[...])


def _mlp_head(x, w1, b1, w2, b2, *, n_blocks=2):
    N, K = x.shape
    Hdim = w1.shape[1]
    Nout = w2.shape[1]
    BN = N // n_blocks
    return pl.pallas_call(
        _mlp_kernel,
        out_shape=jax.ShapeDtypeStruct((N, Nout), jnp.float32),
        grid_spec=pltpu.PrefetchScalarGridSpec(
            num_scalar_prefetch=0,
            grid=(n_blocks,),
            in_specs=[
                pl.BlockSpec((BN, K), lambda i: (i, 0)),
                pl.BlockSpec((K, Hdim), lambda i: (0, 0)),
                pl.BlockSpec((1, Hdim), lambda i: (0, 0)),
                pl.BlockSpec((Hdim, Nout), lambda i: (0, 0)),
                pl.BlockSpec((1, Nout), lambda i: (0, 0)),
            ],
            out_specs=pl.BlockSpec((BN, Nout), lambda i: (i, 0)),
        ),
        compiler_params=pltpu.CompilerParams(
            dimension_semantics=("parallel",),
            vmem_limit_bytes=96 * 1024 * 1024),
    )(x, w1, b1, w2, b2)


def kernel(x_nchw, conv1_w, conv1_shift, conv2_w, conv2_shift,
           conv3_w, conv3_shift, fc1_w, fc1_b, fc2_w, fc2_b):
    N, Cin, H, W = x_nchw.shape

    x = jnp.transpose(x_nchw, (0, 2, 3, 1)).astype(jnp.bfloat16)
    xw = jnp.pad(x, ((0, 0), (0, 0), (1, 1), (0, 0)))
    rowp = jnp.concatenate([xw[:, :, dx:dx + W, :] for dx in range(3)], axis=-1)
    rowp = jnp.pad(rowp, ((0, 0), (1, 1), (0, 0), (0, 0)))

    y = _convs(rowp,
               conv1_w.astype(jnp.bfloat16), conv1_shift,
               conv2_w.astype(jnp.bfloat16), conv2_shift,
               conv3_w.astype(jnp.bfloat16), conv3_shift, B=2)

    flat = y.reshape(N, -1).astype(jnp.bfloat16)
    return _mlp_head(flat, fc1_w.astype(jnp.bfloat16), fc1_b,
                     fc2_w.astype(jnp.bfloat16), fc2_b)


# speedup vs baseline: 1.3108x; 1.3108x over previous
import functools

import jax
import jax.numpy as jnp
from jax.experimental import pallas as pl
from jax.experimental.pallas import tpu as pltpu


def _pool_store(y, out_ref, hp_ref, *, B, H, W, Cout):
    y = y.reshape(B * (H // 2), 2, W, Cout)
    hp = jnp.maximum(y[:, 0], y[:, 1])
    R = B * (H // 2) * W
    hp_ref[...] = hp.reshape(R, Cout)
    pooled = jnp.maximum(hp_ref[pl.ds(0, R // 2, 2), :],
                         hp_ref[pl.ds(1, R // 2, 2), :])
    out_ref[...] = pooled.reshape(B, H // 2, W // 2, Cout).astype(jnp.bfloat16)


def _conv1_kernel(p_ref, w_ref, shift_ref, out_ref, hp_ref, *, B, TH, W, KC, Cout):
    tap = p_ref[...].reshape(B * TH * W, KC)
    y = shift_ref[...].astype(jnp.float32) + jnp.dot(
        tap, w_ref[...], preferred_element_type=jnp.float32)
    y = jnp.maximum(y, 0.0)
    _pool_store(y, out_ref, hp_ref, B=B, H=TH, W=W, Cout=Cout)


def _conv1(patches, w_kc, shift, *, B, TH):
    N, H, W, KC = patches.shape
    Cout = w_kc.shape[-1]

    body = functools.partial(_conv1_kernel, B=B, TH=TH, W=W, KC=KC, Cout=Cout)
    return pl.pallas_call(
        body,
        out_shape=jax.ShapeDtypeStruct((N, H // 2, W // 2, Cout), jnp.bfloat16),
        grid_spec=pltpu.PrefetchScalarGridSpec(
            num_scalar_prefetch=0,
            grid=(N // B, H // TH),
            in_specs=[
                pl.BlockSpec((B, TH, W, KC), lambda n, r: (n, r, 0, 0)),
                pl.BlockSpec((KC, Cout), lambda n, r: (0, 0)),
                pl.BlockSpec((1, Cout), lambda n, r: (0, 0)),
            ],
            out_specs=pl.BlockSpec((B, TH // 2, W // 2, Cout),
                                   lambda n, r: (n, r, 0, 0)),
            scratch_shapes=[
                pltpu.VMEM((B * (TH // 2) * W, Cout), jnp.float32),
            ],
        ),
        compiler_params=pltpu.CompilerParams(
            dimension_semantics=("parallel", "parallel"),
            vmem_limit_bytes=100 * 1024 * 1024),
    )(patches, w_kc, shift)


def _conv_kernel(x_ref, w_ref, shift_ref, out_ref, hp_ref, *, B, H, W, C, Cout):
    y = shift_ref[...].astype(jnp.float32)
    for k in range(9):
        dy, dx = divmod(k, 3)
        tap = x_ref[:, dy:dy + H, dx:dx + W, :].reshape(B * H * W, C)
        y = y + jnp.dot(tap, w_ref[k * C:(k + 1) * C, :],
                        preferred_element_type=jnp.float32)
    y = jnp.maximum(y, 0.0)
    _pool_store(y, out_ref, hp_ref, B=B, H=H, W=W, Cout=Cout)


def _conv_block(x, w_kc, shift, *, B):
    N, H, W, C = x.shape
    Cout = w_kc.shape[-1]
    xp = jnp.pad(x, ((0, 0), (1, 1), (1, 1), (0, 0)))

    body = functools.partial(_conv_kernel, B=B, H=H, W=W, C=C, Cout=Cout)
    return pl.pallas_call(
        body,
        out_shape=jax.ShapeDtypeStruct((N, H // 2, W // 2, Cout), jnp.bfloat16),
        grid_spec=pltpu.PrefetchScalarGridSpec(
            num_scalar_prefetch=0,
            grid=(N // B,),
            in_specs=[
                pl.BlockSpec((B, H + 2, W + 2, C), lambda n: (n, 0, 0, 0)),
                pl.BlockSpec((9 * C, Cout), lambda n: (0, 0)),
                pl.BlockSpec((1, Cout), lambda n: (0, 0)),
            ],
            out_specs=pl.BlockSpec((B, H // 2, W // 2, Cout),
                                   lambda n: (n, 0, 0, 0)),
            scratch_shapes=[
                pltpu.VMEM((B * (H // 2) * W, Cout), jnp.float32),
            ],
        ),
        compiler_params=pltpu.CompilerParams(
            dimension_semantics=("parallel",),
            vmem_limit_bytes=100 * 1024 * 1024),
    )(xp, w_kc, shift)


def _mlp_kernel(x_ref, w1_ref, b1_ref, w2_ref, b2_ref, o_ref):
    h = jnp.dot(x_ref[...], w1_ref[...], preferred_element_type=jnp.float32)
    h = jnp.maximum(h + b1_ref[...], 0.0).astype(jnp.bfloat16)
    o_ref[...] = (jnp.dot(h, w2_ref[...], preferred_element_type=jnp.float32)
                  + b2_ref[...])


def _mlp_head(x, w1, b1, w2, b2, *, n_blocks=2):
    N, K = x.shape
    Hdim = w1.shape[1]
    Nout = w2.shape[1]
    BN = N // n_blocks
    return pl.pallas_call(
        _mlp_kernel,
        out_shape=jax.ShapeDtypeStruct((N, Nout), jnp.float32),
        grid_spec=pltpu.PrefetchScalarGridSpec(
            num_scalar_prefetch=0,
            grid=(n_blocks,),
            in_specs=[
                pl.BlockSpec((BN, K), lambda i: (i, 0)),
                pl.BlockSpec((K, Hdim), lambda i: (0, 0)),
                pl.BlockSpec((1, Hdim), lambda i: (0, 0)),
                pl.BlockSpec((Hdim, Nout), lambda i: (0, 0)),
                pl.BlockSpec((1, Nout), lambda i: (0, 0)),
            ],
            out_specs=pl.BlockSpec((BN, Nout), lambda i: (i, 0)),
        ),
        compiler_params=pltpu.CompilerParams(
            dimension_semantics=("parallel",),
            vmem_limit_bytes=96 * 1024 * 1024),
    )(x, w1, b1, w2, b2)


def kernel(x_nchw, conv1_w, conv1_shift, conv2_w, conv2_shift,
           conv3_w, conv3_shift, fc1_w, fc1_b, fc2_w, fc2_b):
    N, Cin, H, W = x_nchw.shape

    x = jnp.transpose(x_nchw, (0, 2, 3, 1)).astype(jnp.bfloat16)
    xp = jnp.pad(x, ((0, 0), (1, 1), (1, 1), (0, 0)))
    patches = jnp.concatenate(
        [xp[:, ky:ky + H, kx:kx + W, :] for ky in range(3) for kx in range(3)],
        axis=-1)

    y = _conv1(patches, conv1_w.astype(jnp.bfloat16), conv1_shift, B=4, TH=min(32, H))
    y = _conv_block(y, conv2_w.astype(jnp.bfloat16), conv2_shift, B=4)
    y = _conv_block(y, conv3_w.astype(jnp.bfloat16), conv3_shift, B=8)

    flat = y.reshape(N, -1).astype(jnp.bfloat16)
    return _mlp_head(flat, fc1_w.astype(jnp.bfloat16), fc1_b,
                     fc2_w.astype(jnp.bfloat16), fc2_b)
